# Initial kernel scaffold; baseline (speedup 1.0000x reference)
import functools

import jax
import jax.numpy as jnp
from jax.experimental import pallas as pl
from jax.experimental.pallas import tpu as pltpu

_B, _C, _H, _W = 16, 64, 32, 32
_P = _H * _W
_K = 1024


def _vq_body(x_ref, et_ref, e2_ref, q_ref, idx_ref, acc_ref):
    b = pl.program_id(0)
    x_cf = x_ref[0]
    e_t = et_ref[...]
    e2 = e2_ref[...]

    x_pm = jnp.transpose(x_cf, (1, 0))
    x2 = jnp.sum(x_pm * x_pm, axis=1, keepdims=True)
    xe = jax.lax.dot_general(
        x_pm, e_t, (((1,), (0,)), ((), ())),
        preferred_element_type=jnp.float32)
    distances = (x2 + e2) - 2.0 * xe

    idx = jnp.argmax(-distances, axis=1).astype(jnp.int32)
    idx_ref[0, 0, :] = idx

    code_iota = jax.lax.broadcasted_iota(jnp.int32, (_K, _P), 0)
    onehot = (code_iota == idx[None, :]).astype(jnp.float32)
    q_cf = jax.lax.dot_general(
        e_t, onehot, (((1,), (0,)), ((), ())),
        preferred_element_type=jnp.float32,
        precision=jax.lax.Precision.HIGHEST)
    q_ref[0] = q_cf

    diff = q_cf - x_cf

    @pl.when(b == 0)
    def _init():
        acc_ref[0, 0] = 0.0

    acc_ref[0, 0] += jnp.sum(diff * diff)


@functools.partial(jax.jit, static_argnames=())
def kernel(inputs, embedding_weight):
    inputs = inputs.astype(jnp.float32)
    x = inputs.reshape(_B, _C, _P)
    e_t = jnp.transpose(embedding_weight, (1, 0))
    e2 = jnp.sum(embedding_weight.T ** 2, axis=0, keepdims=True)

    q, idx3, acc = pl.pallas_call(
        _vq_body,
        grid=(_B,),
        in_specs=[
            pl.BlockSpec((1, _C, _P), lambda b: (b, 0, 0)),
            pl.BlockSpec((_C, _K), lambda b: (0, 0)),
            pl.BlockSpec((1, _K), lambda b: (0, 0)),
        ],
        out_specs=[
            pl.BlockSpec((1, _C, _P), lambda b: (b, 0, 0)),
            pl.BlockSpec((1, 1, _P), lambda b: (b, 0, 0)),
            pl.BlockSpec((1, 1), lambda b: (0, 0)),
        ],
        out_shape=[
            jax.ShapeDtypeStruct((_B, _C, _P), jnp.float32),
            jax.ShapeDtypeStruct((_B, 1, _P), jnp.int32),
            jax.ShapeDtypeStruct((1, 1), jnp.float32),
        ],
        compiler_params=pltpu.CompilerParams(
            dimension_semantics=("arbitrary",)),
    )(x, e_t, e2)

    quantized_st = q.reshape(_B, _C, _H, _W)
    encoding_indices_v = idx3.reshape(_B, _H, _W)
    loss = acc[0, 0] * (0.25 / (_B * _C * _H * _W))
    encodings_sum = jnp.zeros((256,), dtype=jnp.float32)
    return (quantized_st, loss, encoding_indices_v, encodings_sum,
            embedding_weight)


# speedup vs baseline: 1.2094x; 1.2094x over previous
import functools

import jax
import jax.numpy as jnp
from jax.experimental import pallas as pl
from jax.experimental.pallas import tpu as pltpu

_B, _C, _H, _W = 16, 64, 32, 32
_P = _H * _W
_K = 1024


def _vq_body(x_ref, et_ref, e2_ref, q_ref, idx_ref, acc_ref):
    b = pl.program_id(0)
    x_cf = x_ref[0]
    e_t = et_ref[...]
    e2 = e2_ref[...]

    x_pm = jnp.transpose(x_cf, (1, 0))
    x2 = jnp.sum(x_pm * x_pm, axis=1, keepdims=True)
    xe = jax.lax.dot_general(
        x_pm, e_t, (((1,), (0,)), ((), ())),
        preferred_element_type=jnp.float32)
    distances = (x2 + e2) - 2.0 * xe

    idx = jnp.argmax(-distances, axis=1).astype(jnp.int32)
    idx_ref[0, 0, :] = idx

    code_iota = jax.lax.broadcasted_iota(jnp.int32, (_K, _P), 0)
    onehot = (code_iota == idx[None, :]).astype(jnp.float32)
    q_cf = jax.lax.dot_general(
        e_t, onehot, (((1,), (0,)), ((), ())),
        preferred_element_type=jnp.float32,
        precision=jax.lax.Precision.HIGHEST)
    q_ref[0] = q_cf

    diff = q_cf - x_cf

    @pl.when(b == 0)
    def _init():
        acc_ref[...] = jnp.zeros((1, 1), jnp.float32)

    acc_ref[...] += jnp.sum(diff * diff).reshape(1, 1)


@functools.partial(jax.jit, static_argnames=())
def kernel(inputs, embedding_weight):
    inputs = inputs.astype(jnp.float32)
    x = inputs.reshape(_B, _C, _P)
    e_t = jnp.transpose(embedding_weight, (1, 0))
    e2 = jnp.sum(embedding_weight.T ** 2, axis=0, keepdims=True)

    q, idx3, acc = pl.pallas_call(
        _vq_body,
        grid=(_B,),
        in_specs=[
            pl.BlockSpec((1, _C, _P), lambda b: (b, 0, 0)),
            pl.BlockSpec((_C, _K), lambda b: (0, 0)),
            pl.BlockSpec((1, _K), lambda b: (0, 0)),
        ],
        out_specs=[
            pl.BlockSpec((1, _C, _P), lambda b: (b, 0, 0)),
            pl.BlockSpec((1, 1, _P), lambda b: (b, 0, 0)),
            pl.BlockSpec((1, 1), lambda b: (0, 0)),
        ],
        out_shape=[
            jax.ShapeDtypeStruct((_B, _C, _P), jnp.float32),
            jax.ShapeDtypeStruct((_B, 1, _P), jnp.int32),
            jax.ShapeDtypeStruct((1, 1), jnp.float32),
        ],
        compiler_params=pltpu.CompilerParams(
            dimension_semantics=("arbitrary",)),
    )(x, e_t, e2)

    quantized_st = q.reshape(_B, _C, _H, _W)
    encoding_indices_v = idx3.reshape(_B, _H, _W)
    loss = acc[0, 0] * (0.25 / (_B * _C * _H * _W))
    encodings_sum = jnp.zeros((256,), dtype=jnp.float32)
    return (quantized_st, loss, encoding_indices_v, encodings_sum,
            embedding_weight)
